# Initial kernel scaffold; baseline (speedup 1.0000x reference)
import jax
import jax.numpy as jnp
from jax.experimental import pallas as pl


def kernel(x_masked):
    raise NotImplementedError("write your pallas kernel here")


# speedup vs baseline: 4.0617x; 4.0617x over previous
import functools

import jax
import jax.numpy as jnp
from jax import lax
from jax.experimental import pallas as pl
from jax.experimental.pallas import tpu as pltpu
from jax.experimental.pallas import tpu_sc as plsc

B, T, D = 8, 512, 64
L = 16
NBLK = D // L


def _tec_body(x_hbm, out_hbm, slab, pv_buf, pi_buf):
    cid = lax.axis_index("c")
    sid = lax.axis_index("s")
    wid = sid * 2 + cid
    b = wid // NBLK
    dcol = (wid % NBLK) * L

    pltpu.sync_copy(x_hbm.at[b, :, pl.ds(dcol, L)], slab)

    zero = jnp.zeros((L,), jnp.float32)
    neg1 = jnp.full((L,), -1, jnp.int32)

    def fwd(t, carry):
        pv, pi = carry
        v = slab[t]
        nz = v != 0.0
        pv = jnp.where(nz, v, pv)
        pi = jnp.where(nz, jnp.full((L,), t, jnp.int32), pi)
        pv_buf[t] = pv
        pi_buf[t] = pi
        return pv, pi

    lax.fori_loop(0, T, fwd, (zero, neg1), unroll=2)

    big = jnp.full((L,), T, jnp.int32)

    def bwd(k, carry):
        nv, ni = carry
        t = T - 1 - k
        v = slab[t]
        nz = v != 0.0
        nv = jnp.where(nz, v, nv)
        ni = jnp.where(nz, jnp.full((L,), t, jnp.int32), ni)
        pv = pv_buf[t]
        pi = pi_buf[t]
        n = ni - pi - 1
        i = jnp.full((L,), t, jnp.int32) - pi - 1
        denom = jnp.maximum(n - 1, 1).astype(jnp.float32)
        frac = jnp.where(n > 1, i.astype(jnp.float32) / denom, zero)
        val = pv + frac * (nv - pv)
        fill = jnp.where(pi >= 0, jnp.where(ni < T, val, v), v)
        slab[t] = jnp.where(nz, v, fill)
        return nv, ni

    lax.fori_loop(0, T, bwd, (zero, big), unroll=2)

    pltpu.sync_copy(slab, out_hbm.at[b, :, pl.ds(dcol, L)])


@jax.jit
def kernel(x_masked):
    mesh = plsc.VectorSubcoreMesh(core_axis_name="c", subcore_axis_name="s")
    run = pl.kernel(
        _tec_body,
        out_type=jax.ShapeDtypeStruct((B, T, D), jnp.float32),
        mesh=mesh,
        scratch_types=[
            pltpu.VMEM((T, L), jnp.float32),
            pltpu.VMEM((T, L), jnp.float32),
            pltpu.VMEM((T, L), jnp.int32),
        ],
        compiler_params=pltpu.CompilerParams(use_tc_tiling_on_sc=False),
    )
    return run(x_masked)
